# baseline (speedup 1.0000x reference)
import functools

import jax
import jax.numpy as jnp
from jax import lax
from jax.experimental import pallas as pl
from jax.experimental.pallas import tpu as pltpu
from jax.experimental.pallas import tpu_sc as plsc

_B = 8
_N = 8192
_G = 512
_K = 32
_BIG = 1 << 30


def _fps_body(xT_ref, cT_ref, dist_ref):
    x0 = xT_ref[0]
    x1 = xT_ref[1]
    x2 = xT_ref[2]
    lane = lax.broadcasted_iota(jnp.int32, (_B, _N), 1)
    lane_g = lax.broadcasted_iota(jnp.int32, (_B, _G), 1)
    dist_ref[...] = jnp.full((_B, _N), 1e10, jnp.float32)

    def step(j, far):
        m = lane == far
        c0 = jnp.sum(jnp.where(m, x0, 0.0), axis=1, keepdims=True)
        c1 = jnp.sum(jnp.where(m, x1, 0.0), axis=1, keepdims=True)
        c2 = jnp.sum(jnp.where(m, x2, 0.0), axis=1, keepdims=True)
        sel = lane_g == j
        cT_ref[0] = jnp.where(sel, c0, cT_ref[0])
        cT_ref[1] = jnp.where(sel, c1, cT_ref[1])
        cT_ref[2] = jnp.where(sel, c2, cT_ref[2])
        d0 = x0 - c0
        d1 = x1 - c1
        d2 = x2 - c2
        d = (d0 * d0 + d2 * d2) + d1 * d1
        dist = jnp.minimum(dist_ref[...], d)
        dist_ref[...] = dist
        mx = jnp.max(dist, axis=1, keepdims=True)
        cand = jnp.where(dist == mx, lane, _BIG)
        return jnp.min(cand, axis=1, keepdims=True)

    far0 = jnp.zeros((_B, 1), jnp.int32)
    lax.fori_loop(0, _G, step, far0)


def _fps(xT):
    return pl.pallas_call(
        _fps_body,
        out_shape=jax.ShapeDtypeStruct((3, _B, _G), jnp.float32),
        scratch_shapes=[pltpu.VMEM((_B, _N), jnp.float32)],
    )(xT)


_GB = 64


def _knn_body(c_ref, xT_ref, idx_ref):
    b = pl.program_id(0)
    c = c_ref[0]
    xt = xT_ref[0]
    cx = lax.dot_general(c, xt, (((1,), (0,)), ((), ())))
    c0 = c[:, 0:1]
    c1 = c[:, 1:2]
    c2 = c[:, 2:3]
    csq = (c0 * c0 + c1 * c1) + c2 * c2
    x0 = xt[0:1, :]
    x1 = xt[1:2, :]
    x2 = xt[2:3, :]
    xsq = (x0 * x0 + x1 * x1) + x2 * x2
    d = (csq + xsq) - 2.0 * cx

    lane_k = lax.broadcasted_iota(jnp.int32, (_GB, _K), 1)
    bias = b * _N
    n_ch = _N // 128
    iota128 = lax.broadcasted_iota(jnp.int32, (_GB, 128), 1)

    def comb(a, p):
        av, ai = a
        bv, bi = p
        t = (bv < av) | ((bv == av) & (bi < ai))
        return jnp.where(t, bv, av), jnp.where(t, bi, ai)

    def rnd(j, carry):
        d, acc = carry
        pairs = [(d[:, k * 128:(k + 1) * 128], iota128 + k * 128)
                 for k in range(n_ch)]
        while len(pairs) > 1:
            pairs = [comb(pairs[i], pairs[i + 1])
                     for i in range(0, len(pairs), 2)]
        v128, i128 = pairs[0]
        mn = jnp.min(v128, axis=1, keepdims=True)
        am = jnp.min(jnp.where(v128 == mn, i128, _BIG), axis=1, keepdims=True)
        acc = jnp.where(lane_k == j, am + bias, acc)
        d = jnp.concatenate(
            [jnp.where(iota128 + k * 128 == am, jnp.inf, d[:, k * 128:(k + 1) * 128])
             for k in range(n_ch)], axis=1)
        return d, acc

    _, acc = lax.fori_loop(0, _K, rnd, (d, jnp.zeros((_GB, _K), jnp.int32)))
    idx_ref[0] = acc


def _knn(cL, xT2):
    return pl.pallas_call(
        _knn_body,
        grid=(_B, _G // _GB),
        in_specs=[
            pl.BlockSpec((1, _GB, 3), lambda b, g: (b, g, 0)),
            pl.BlockSpec((1, 3, _N), lambda b, g: (b, 0, 0)),
        ],
        out_specs=pl.BlockSpec((1, _GB, _K), lambda b, g: (b, g, 0)),
        out_shape=jax.ShapeDtypeStruct((_B, _G, _K), jnp.int32),
    )(cL, xT2)


_NW = 32
_RPW = (_B * _G * _K) // _NW
_GPW = _RPW // _K


def _gather_body(xyzp_hbm, gidx_hbm, cpad_hbm, out_hbm, idx_v, rows_v,
                 cen_v, sem):
    wid = lax.axis_index("s") * 2 + lax.axis_index("c")
    row0 = wid * _RPW
    pltpu.sync_copy(gidx_hbm.at[pl.ds(row0, _RPW)], idx_v)
    pltpu.sync_copy(cpad_hbm.at[pl.ds(wid * _GPW, _GPW)], cen_v)
    pltpu.async_copy(xyzp_hbm.at[idx_v], rows_v, sem).wait()

    def per_group(g, _):
        ctile = cen_v[g]
        for v in range(_K):
            r = g * _K + v
            rows_v[r] = rows_v[r] - ctile
        return 0

    lax.fori_loop(0, _GPW, per_group, 0)
    pltpu.sync_copy(rows_v, out_hbm.at[pl.ds(row0, _RPW)])


@functools.lru_cache(maxsize=1)
def _gather_sc():
    return pl.kernel(
        _gather_body,
        out_type=jax.ShapeDtypeStruct((_B * _G * _K, 16), jnp.float32),
        mesh=plsc.VectorSubcoreMesh(core_axis_name="c", subcore_axis_name="s"),
        compiler_params=pltpu.CompilerParams(use_tc_tiling_on_sc=False),
        scratch_types=[
            pltpu.VMEM((_RPW,), jnp.int32),
            pltpu.VMEM((_RPW, 16), jnp.float32),
            pltpu.VMEM((_GPW, 16), jnp.float32),
            pltpu.SemaphoreType.DMA,
        ],
    )


def kernel(xyz):
    xT = xyz.transpose(2, 0, 1)
    cT = _fps(xT)
    center = cT.transpose(1, 2, 0)
    gidx = _knn(center, xyz.transpose(0, 2, 1))
    xyzp = jnp.pad(xyz.reshape(_B * _N, 3), ((0, 0), (0, 13)))
    cpad = jnp.pad(center.reshape(_B * _G, 3), ((0, 0), (0, 13)))
    out = _gather_sc()(xyzp, gidx.reshape(-1), cpad)
    neighborhood = out.reshape(_B, _G, _K, 16)[..., :3]
    return (neighborhood, center)


# speedup vs baseline: 1.3071x; 1.3071x over previous
import functools

import jax
import jax.numpy as jnp
from jax import lax
from jax.experimental import pallas as pl
from jax.experimental.pallas import tpu as pltpu
from jax.experimental.pallas import tpu_sc as plsc

_B = 8
_N = 8192
_G = 512
_K = 32
_BIG = 1 << 30


def _fps_body(xT_ref, cT_ref, dist_ref):
    x0 = xT_ref[0]
    x1 = xT_ref[1]
    x2 = xT_ref[2]
    lane = lax.broadcasted_iota(jnp.int32, (_B, _N), 1)
    lane_g = lax.broadcasted_iota(jnp.int32, (_B, _G), 1)
    dist_ref[...] = jnp.full((_B, _N), 1e10, jnp.float32)

    def step(j, far):
        m = lane == far
        c0 = jnp.sum(jnp.where(m, x0, 0.0), axis=1, keepdims=True)
        c1 = jnp.sum(jnp.where(m, x1, 0.0), axis=1, keepdims=True)
        c2 = jnp.sum(jnp.where(m, x2, 0.0), axis=1, keepdims=True)
        sel = lane_g == j
        cT_ref[0] = jnp.where(sel, c0, cT_ref[0])
        cT_ref[1] = jnp.where(sel, c1, cT_ref[1])
        cT_ref[2] = jnp.where(sel, c2, cT_ref[2])
        d0 = x0 - c0
        d1 = x1 - c1
        d2 = x2 - c2
        d = (d0 * d0 + d2 * d2) + d1 * d1
        dist = jnp.minimum(dist_ref[...], d)
        dist_ref[...] = dist
        mx = jnp.max(dist, axis=1, keepdims=True)
        cand = jnp.where(dist == mx, lane, _BIG)
        return jnp.min(cand, axis=1, keepdims=True)

    far0 = jnp.zeros((_B, 1), jnp.int32)
    lax.fori_loop(0, _G, step, far0)


def _fps(xT):
    return pl.pallas_call(
        _fps_body,
        out_shape=jax.ShapeDtypeStruct((3, _B, _G), jnp.float32),
        scratch_shapes=[pltpu.VMEM((_B, _N), jnp.float32)],
    )(xT)


_GB = 64


def _knn_body(c_ref, xT_ref, idx_ref):
    b = pl.program_id(0)
    c = c_ref[0]
    xt = xT_ref[0]
    cx = lax.dot_general(c, xt, (((1,), (0,)), ((), ())))
    c0 = c[:, 0:1]
    c1 = c[:, 1:2]
    c2 = c[:, 2:3]
    csq = (c0 * c0 + c1 * c1) + c2 * c2
    x0 = xt[0:1, :]
    x1 = xt[1:2, :]
    x2 = xt[2:3, :]
    xsq = (x0 * x0 + x1 * x1) + x2 * x2
    d = (csq + xsq) - 2.0 * cx

    bias = b * _N
    n_ch = _N // 128
    iota128 = lax.broadcasted_iota(jnp.int32, (_GB, 128), 1)

    def comb(a, p):
        av, ai = a
        bv, bi = p
        t = bv < av
        return jnp.where(t, bv, av), jnp.where(t, bi, ai)

    chunks = [d[:, k * 128:(k + 1) * 128] for k in range(n_ch)]
    acc = []
    for _ in range(_K):
        pairs = [(chunks[k], iota128 + k * 128) for k in range(n_ch)]
        while len(pairs) > 1:
            pairs = [comb(pairs[i], pairs[i + 1])
                     for i in range(0, len(pairs), 2)]
        v128, i128 = pairs[0]
        mn = jnp.min(v128, axis=1, keepdims=True)
        am = jnp.min(jnp.where(v128 == mn, i128, _BIG), axis=1, keepdims=True)
        acc.append(am + bias)
        chunks = [jnp.where(iota128 == am - k * 128, jnp.inf, chunks[k])
                  for k in range(n_ch)]
    idx_ref[0] = jnp.concatenate(acc, axis=1)


def _knn(cL, xT2):
    return pl.pallas_call(
        _knn_body,
        grid=(_B, _G // _GB),
        in_specs=[
            pl.BlockSpec((1, _GB, 3), lambda b, g: (b, g, 0)),
            pl.BlockSpec((1, 3, _N), lambda b, g: (b, 0, 0)),
        ],
        out_specs=pl.BlockSpec((1, _GB, _K), lambda b, g: (b, g, 0)),
        out_shape=jax.ShapeDtypeStruct((_B, _G, _K), jnp.int32),
    )(cL, xT2)


_NW = 32
_RPW = (_B * _G * _K) // _NW
_GPW = _RPW // _K


def _gather_body(xyzp_hbm, gidx_hbm, cpad_hbm, out_hbm, idx_v, rows_v,
                 cen_v, sem):
    wid = lax.axis_index("s") * 2 + lax.axis_index("c")
    row0 = wid * _RPW
    pltpu.sync_copy(gidx_hbm.at[pl.ds(row0, _RPW)], idx_v)
    pltpu.sync_copy(cpad_hbm.at[pl.ds(wid * _GPW, _GPW)], cen_v)
    pltpu.async_copy(xyzp_hbm.at[idx_v], rows_v, sem).wait()

    def per_group(g, _):
        ctile = cen_v[g]
        for v in range(_K):
            r = g * _K + v
            rows_v[r] = rows_v[r] - ctile
        return 0

    lax.fori_loop(0, _GPW, per_group, 0)
    pltpu.sync_copy(rows_v, out_hbm.at[pl.ds(row0, _RPW)])


@functools.lru_cache(maxsize=1)
def _gather_sc():
    return pl.kernel(
        _gather_body,
        out_type=jax.ShapeDtypeStruct((_B * _G * _K, 16), jnp.float32),
        mesh=plsc.VectorSubcoreMesh(core_axis_name="c", subcore_axis_name="s"),
        compiler_params=pltpu.CompilerParams(use_tc_tiling_on_sc=False),
        scratch_types=[
            pltpu.VMEM((_RPW,), jnp.int32),
            pltpu.VMEM((_RPW, 16), jnp.float32),
            pltpu.VMEM((_GPW, 16), jnp.float32),
            pltpu.SemaphoreType.DMA,
        ],
    )


def kernel(xyz):
    xT = xyz.transpose(2, 0, 1)
    cT = _fps(xT)
    center = cT.transpose(1, 2, 0)
    gidx = _knn(center, xyz.transpose(0, 2, 1))
    xyzp = jnp.pad(xyz.reshape(_B * _N, 3), ((0, 0), (0, 13)))
    cpad = jnp.pad(center.reshape(_B * _G, 3), ((0, 0), (0, 13)))
    out = _gather_sc()(xyzp, gidx.reshape(-1), cpad)
    neighborhood = out.reshape(_B, _G, _K, 16)[..., :3]
    return (neighborhood, center)


# speedup vs baseline: 1.4563x; 1.1141x over previous
import functools

import jax
import jax.numpy as jnp
from jax import lax
from jax.experimental import pallas as pl
from jax.experimental.pallas import tpu as pltpu
from jax.experimental.pallas import tpu_sc as plsc

_B = 8
_N = 8192
_G = 512
_K = 32
_BIG = 1 << 30


def _fps_body(xT_ref, cT_ref, dist_ref):
    x0 = xT_ref[0]
    x1 = xT_ref[1]
    x2 = xT_ref[2]
    lane = lax.broadcasted_iota(jnp.int32, (_B, _N), 1)
    lane_g = lax.broadcasted_iota(jnp.int32, (_B, _G), 1)
    dist_ref[...] = jnp.full((_B, _N), 1e10, jnp.float32)

    def step(j, far):
        m = lane == far
        c0 = jnp.sum(jnp.where(m, x0, 0.0), axis=1, keepdims=True)
        c1 = jnp.sum(jnp.where(m, x1, 0.0), axis=1, keepdims=True)
        c2 = jnp.sum(jnp.where(m, x2, 0.0), axis=1, keepdims=True)
        sel = lane_g == j
        cT_ref[0] = jnp.where(sel, c0, cT_ref[0])
        cT_ref[1] = jnp.where(sel, c1, cT_ref[1])
        cT_ref[2] = jnp.where(sel, c2, cT_ref[2])
        d0 = x0 - c0
        d1 = x1 - c1
        d2 = x2 - c2
        d = (d0 * d0 + d2 * d2) + d1 * d1
        dist = jnp.minimum(dist_ref[...], d)
        dist_ref[...] = dist
        mx = jnp.max(dist, axis=1, keepdims=True)
        cand = jnp.where(dist == mx, lane, _BIG)
        return jnp.min(cand, axis=1, keepdims=True)

    far0 = jnp.zeros((_B, 1), jnp.int32)
    lax.fori_loop(0, _G, step, far0)


def _fps(xT):
    return pl.pallas_call(
        _fps_body,
        out_shape=jax.ShapeDtypeStruct((3, _B, _G), jnp.float32),
        scratch_shapes=[pltpu.VMEM((_B, _N), jnp.float32)],
    )(xT)


_GB = 64


def _knn_body(c_ref, xT_ref, idx_ref):
    b = pl.program_id(0)
    c = c_ref[0]
    xt = xT_ref[0]
    cx = lax.dot_general(c, xt, (((1,), (0,)), ((), ())))
    c0 = c[:, 0:1]
    c1 = c[:, 1:2]
    c2 = c[:, 2:3]
    csq = (c0 * c0 + c1 * c1) + c2 * c2
    x0 = xt[0:1, :]
    x1 = xt[1:2, :]
    x2 = xt[2:3, :]
    xsq = (x0 * x0 + x1 * x1) + x2 * x2
    d = (csq + xsq) - 2.0 * cx

    bias = b * _N
    lane = lax.broadcasted_iota(jnp.int32, (_GB, _N), 1)
    acc = []
    mn = jnp.min(d, axis=1, keepdims=True)
    for j in range(_K):
        cand = jnp.where(d == mn, lane, _BIG)
        am = jnp.min(cand, axis=1, keepdims=True)
        acc.append(am + bias)
        if j + 1 < _K:
            d = jnp.where(lane == am, jnp.inf, d)
            mn = jnp.min(d, axis=1, keepdims=True)
    idx_ref[0] = jnp.concatenate(acc, axis=1)


def _knn(cL, xT2):
    return pl.pallas_call(
        _knn_body,
        grid=(_B, _G // _GB),
        in_specs=[
            pl.BlockSpec((1, _GB, 3), lambda b, g: (b, g, 0)),
            pl.BlockSpec((1, 3, _N), lambda b, g: (b, 0, 0)),
        ],
        out_specs=pl.BlockSpec((1, _GB, _K), lambda b, g: (b, g, 0)),
        out_shape=jax.ShapeDtypeStruct((_B, _G, _K), jnp.int32),
    )(cL, xT2)


_NW = 32
_RPW = (_B * _G * _K) // _NW
_GPW = _RPW // _K


def _gather_body(xyzp_hbm, gidx_hbm, cpad_hbm, out_hbm, idx_v, rows_v,
                 cen_v, sem):
    wid = lax.axis_index("s") * 2 + lax.axis_index("c")
    row0 = wid * _RPW
    pltpu.sync_copy(gidx_hbm.at[pl.ds(row0, _RPW)], idx_v)
    pltpu.sync_copy(cpad_hbm.at[pl.ds(wid * _GPW, _GPW)], cen_v)
    pltpu.async_copy(xyzp_hbm.at[idx_v], rows_v, sem).wait()

    def per_group(g, _):
        ctile = cen_v[g]
        for v in range(_K):
            r = g * _K + v
            rows_v[r] = rows_v[r] - ctile
        return 0

    lax.fori_loop(0, _GPW, per_group, 0)
    pltpu.sync_copy(rows_v, out_hbm.at[pl.ds(row0, _RPW)])


@functools.lru_cache(maxsize=1)
def _gather_sc():
    return pl.kernel(
        _gather_body,
        out_type=jax.ShapeDtypeStruct((_B * _G * _K, 16), jnp.float32),
        mesh=plsc.VectorSubcoreMesh(core_axis_name="c", subcore_axis_name="s"),
        compiler_params=pltpu.CompilerParams(use_tc_tiling_on_sc=False),
        scratch_types=[
            pltpu.VMEM((_RPW,), jnp.int32),
            pltpu.VMEM((_RPW, 16), jnp.float32),
            pltpu.VMEM((_GPW, 16), jnp.float32),
            pltpu.SemaphoreType.DMA,
        ],
    )


def kernel(xyz):
    xT = xyz.transpose(2, 0, 1)
    cT = _fps(xT)
    center = cT.transpose(1, 2, 0)
    gidx = _knn(center, xyz.transpose(0, 2, 1))
    xyzp = jnp.pad(xyz.reshape(_B * _N, 3), ((0, 0), (0, 13)))
    cpad = jnp.pad(center.reshape(_B * _G, 3), ((0, 0), (0, 13)))
    out = _gather_sc()(xyzp, gidx.reshape(-1), cpad)
    neighborhood = out.reshape(_B, _G, _K, 16)[..., :3]
    return (neighborhood, center)


# speedup vs baseline: 1.4596x; 1.0023x over previous
import functools

import jax
import jax.numpy as jnp
from jax import lax
from jax.experimental import pallas as pl
from jax.experimental.pallas import tpu as pltpu
from jax.experimental.pallas import tpu_sc as plsc

_B = 8
_N = 8192
_G = 512
_K = 32
_BIG = 1 << 30


def _fps_body(xT_ref, cT_ref, dist_ref):
    x0 = xT_ref[0]
    x1 = xT_ref[1]
    x2 = xT_ref[2]
    lane = lax.broadcasted_iota(jnp.int32, (_B, _N), 1)
    lane_g = lax.broadcasted_iota(jnp.int32, (_B, _G), 1)
    dist_ref[...] = jnp.full((_B, _N), 1e10, jnp.float32)

    def step(j, far):
        m = lane == far
        c0 = jnp.sum(jnp.where(m, x0, 0.0), axis=1, keepdims=True)
        c1 = jnp.sum(jnp.where(m, x1, 0.0), axis=1, keepdims=True)
        c2 = jnp.sum(jnp.where(m, x2, 0.0), axis=1, keepdims=True)
        sel = lane_g == j
        cT_ref[0] = jnp.where(sel, c0, cT_ref[0])
        cT_ref[1] = jnp.where(sel, c1, cT_ref[1])
        cT_ref[2] = jnp.where(sel, c2, cT_ref[2])
        d0 = x0 - c0
        d1 = x1 - c1
        d2 = x2 - c2
        d = (d0 * d0 + d2 * d2) + d1 * d1
        dist = jnp.minimum(dist_ref[...], d)
        dist_ref[...] = dist
        mx = jnp.max(dist, axis=1, keepdims=True)
        cand = jnp.where(dist == mx, lane, _BIG)
        return jnp.min(cand, axis=1, keepdims=True)

    far0 = jnp.zeros((_B, 1), jnp.int32)
    lax.fori_loop(0, _G, step, far0)


def _fps(xT):
    return pl.pallas_call(
        _fps_body,
        out_shape=jax.ShapeDtypeStruct((3, _B, _G), jnp.float32),
        scratch_shapes=[pltpu.VMEM((_B, _N), jnp.float32)],
    )(xT)


_GB = 64


def _knn_body(c_ref, xT_ref, idx_ref):
    b = pl.program_id(0)
    c = c_ref[0]
    xt = xT_ref[:, b, :]
    cx = lax.dot_general(c, xt, (((1,), (0,)), ((), ())))
    c0 = c[:, 0:1]
    c1 = c[:, 1:2]
    c2 = c[:, 2:3]
    csq = (c0 * c0 + c1 * c1) + c2 * c2
    x0 = xt[0:1, :]
    x1 = xt[1:2, :]
    x2 = xt[2:3, :]
    xsq = (x0 * x0 + x1 * x1) + x2 * x2
    d = (csq + xsq) - 2.0 * cx

    bias = b * _N
    lane = lax.broadcasted_iota(jnp.int32, (_GB, _N), 1)
    acc = []
    mn = jnp.min(d, axis=1, keepdims=True)
    for j in range(_K):
        cand = jnp.where(d == mn, lane, _BIG)
        am = jnp.min(cand, axis=1, keepdims=True)
        acc.append(am + bias)
        if j + 1 < _K:
            d = jnp.where(lane == am, jnp.inf, d)
            mn = jnp.min(d, axis=1, keepdims=True)
    idx_ref[0] = jnp.concatenate(acc, axis=1)


def _knn(cL, xT2):
    return pl.pallas_call(
        _knn_body,
        grid=(_B, _G // _GB),
        in_specs=[
            pl.BlockSpec((1, _GB, 3), lambda b, g: (b, g, 0)),
            pl.BlockSpec((3, _B, _N), lambda b, g: (0, 0, 0)),
        ],
        out_specs=pl.BlockSpec((1, _GB, _K), lambda b, g: (b, g, 0)),
        out_shape=jax.ShapeDtypeStruct((_B, _G, _K), jnp.int32),
    )(cL, xT2)


_NW = 32
_RPW = (_B * _G * _K) // _NW
_GPW = _RPW // _K


def _gather_body(xyzp_hbm, gidx_hbm, cpad_hbm, out_hbm, idx_v, rows_v,
                 cen_v, sem):
    wid = lax.axis_index("s") * 2 + lax.axis_index("c")
    row0 = wid * _RPW
    pltpu.sync_copy(gidx_hbm.at[pl.ds(row0, _RPW)], idx_v)
    pltpu.sync_copy(cpad_hbm.at[pl.ds(wid * _GPW, _GPW)], cen_v)
    pltpu.async_copy(xyzp_hbm.at[idx_v], rows_v, sem).wait()

    def per_group(g, _):
        ctile = cen_v[g]
        for v in range(_K):
            r = g * _K + v
            rows_v[r] = rows_v[r] - ctile
        return 0

    lax.fori_loop(0, _GPW, per_group, 0)
    pltpu.sync_copy(rows_v, out_hbm.at[pl.ds(row0, _RPW)])


@functools.lru_cache(maxsize=1)
def _gather_sc():
    return pl.kernel(
        _gather_body,
        out_type=jax.ShapeDtypeStruct((_B * _G * _K, 16), jnp.float32),
        mesh=plsc.VectorSubcoreMesh(core_axis_name="c", subcore_axis_name="s"),
        compiler_params=pltpu.CompilerParams(use_tc_tiling_on_sc=False),
        scratch_types=[
            pltpu.VMEM((_RPW,), jnp.int32),
            pltpu.VMEM((_RPW, 16), jnp.float32),
            pltpu.VMEM((_GPW, 16), jnp.float32),
            pltpu.SemaphoreType.DMA,
        ],
    )


def kernel(xyz):
    xT = xyz.transpose(2, 0, 1)
    cT = _fps(xT)
    center = cT.transpose(1, 2, 0)
    gidx = _knn(center, xT)
    xyzp = jnp.pad(xyz.reshape(_B * _N, 3), ((0, 0), (0, 13)))
    cpad = jnp.pad(center.reshape(_B * _G, 3), ((0, 0), (0, 13)))
    out = _gather_sc()(xyzp, gidx.reshape(-1), cpad)
    neighborhood = out.reshape(_B, _G, _K, 16)[..., :3]
    return (neighborhood, center)


# speedup vs baseline: 1.5680x; 1.0742x over previous
import functools

import jax
import jax.numpy as jnp
from jax import lax
from jax.experimental import pallas as pl
from jax.experimental.pallas import tpu as pltpu
from jax.experimental.pallas import tpu_sc as plsc

_B = 8
_N = 8192
_G = 512
_K = 32
_BIG = 1 << 30


def _fps_body(xT_ref, cT_ref, dist_ref):
    x0 = xT_ref[0]
    x1 = xT_ref[1]
    x2 = xT_ref[2]
    lane = lax.broadcasted_iota(jnp.int32, (_B, _N), 1)
    lane_g = lax.broadcasted_iota(jnp.int32, (_B, _G), 1)
    dist_ref[...] = jnp.full((_B, _N), 1e10, jnp.float32)

    def step(j, far):
        m = lane == far
        c0 = jnp.sum(jnp.where(m, x0, 0.0), axis=1, keepdims=True)
        c1 = jnp.sum(jnp.where(m, x1, 0.0), axis=1, keepdims=True)
        c2 = jnp.sum(jnp.where(m, x2, 0.0), axis=1, keepdims=True)
        sel = lane_g == j
        cT_ref[0] = jnp.where(sel, c0, cT_ref[0])
        cT_ref[1] = jnp.where(sel, c1, cT_ref[1])
        cT_ref[2] = jnp.where(sel, c2, cT_ref[2])
        d0 = x0 - c0
        d1 = x1 - c1
        d2 = x2 - c2
        d = (d0 * d0 + d2 * d2) + d1 * d1
        dist = jnp.minimum(dist_ref[...], d)
        dist_ref[...] = dist
        mx = jnp.max(dist, axis=1, keepdims=True)
        cand = jnp.where(dist == mx, lane, _BIG)
        return jnp.min(cand, axis=1, keepdims=True)

    far0 = jnp.zeros((_B, 1), jnp.int32)
    lax.fori_loop(0, _G, step, far0)


def _fps(xT):
    return pl.pallas_call(
        _fps_body,
        out_shape=jax.ShapeDtypeStruct((3, _B, _G), jnp.float32),
        scratch_shapes=[pltpu.VMEM((_B, _N), jnp.float32)],
    )(xT)


_GB = 512


def _knn_body(c_ref, xT_ref, idx_ref):
    b = pl.program_id(0)
    c = c_ref[0]
    xt = xT_ref[:, b, :]
    cx = lax.dot_general(c, xt, (((1,), (0,)), ((), ())))
    c0 = c[:, 0:1]
    c1 = c[:, 1:2]
    c2 = c[:, 2:3]
    csq = (c0 * c0 + c1 * c1) + c2 * c2
    x0 = xt[0:1, :]
    x1 = xt[1:2, :]
    x2 = xt[2:3, :]
    xsq = (x0 * x0 + x1 * x1) + x2 * x2
    d = (csq + xsq) - 2.0 * cx

    bias = b * _N
    lane = lax.broadcasted_iota(jnp.int32, (_GB, _N), 1)
    acc = []
    mn = jnp.min(d, axis=1, keepdims=True)
    for j in range(_K):
        cand = jnp.where(d == mn, lane, _BIG)
        am = jnp.min(cand, axis=1, keepdims=True)
        acc.append(am + bias)
        if j + 1 < _K:
            d = jnp.where(lane == am, jnp.inf, d)
            mn = jnp.min(d, axis=1, keepdims=True)
    idx_ref[0] = jnp.concatenate(acc, axis=1)


def _knn(cL, xT2):
    return pl.pallas_call(
        _knn_body,
        grid=(_B, _G // _GB),
        in_specs=[
            pl.BlockSpec((1, _GB, 3), lambda b, g: (b, g, 0)),
            pl.BlockSpec((3, _B, _N), lambda b, g: (0, 0, 0)),
        ],
        out_specs=pl.BlockSpec((1, _GB, _K), lambda b, g: (b, g, 0)),
        out_shape=jax.ShapeDtypeStruct((_B, _G, _K), jnp.int32),
    )(cL, xT2)


_NW = 32
_RPW = (_B * _G * _K) // _NW
_GPW = _RPW // _K


def _gather_body(xyzp_hbm, gidx_hbm, cpad_hbm, out_hbm, idx_v, rows_v,
                 cen_v, sem):
    wid = lax.axis_index("s") * 2 + lax.axis_index("c")
    row0 = wid * _RPW
    pltpu.sync_copy(gidx_hbm.at[pl.ds(row0, _RPW)], idx_v)
    pltpu.sync_copy(cpad_hbm.at[pl.ds(wid * _GPW, _GPW)], cen_v)
    pltpu.async_copy(xyzp_hbm.at[idx_v], rows_v, sem).wait()

    def per_group(g, _):
        ctile = cen_v[g]
        for v in range(_K):
            r = g * _K + v
            rows_v[r] = rows_v[r] - ctile
        return 0

    lax.fori_loop(0, _GPW, per_group, 0)
    pltpu.sync_copy(rows_v, out_hbm.at[pl.ds(row0, _RPW)])


@functools.lru_cache(maxsize=1)
def _gather_sc():
    return pl.kernel(
        _gather_body,
        out_type=jax.ShapeDtypeStruct((_B * _G * _K, 16), jnp.float32),
        mesh=plsc.VectorSubcoreMesh(core_axis_name="c", subcore_axis_name="s"),
        compiler_params=pltpu.CompilerParams(use_tc_tiling_on_sc=False),
        scratch_types=[
            pltpu.VMEM((_RPW,), jnp.int32),
            pltpu.VMEM((_RPW, 16), jnp.float32),
            pltpu.VMEM((_GPW, 16), jnp.float32),
            pltpu.SemaphoreType.DMA,
        ],
    )


def kernel(xyz):
    xT = xyz.transpose(2, 0, 1)
    cT = _fps(xT)
    center = cT.transpose(1, 2, 0)
    gidx = _knn(center, xT)
    xyzp = jnp.pad(xyz.reshape(_B * _N, 3), ((0, 0), (0, 13)))
    cpad = jnp.pad(center.reshape(_B * _G, 3), ((0, 0), (0, 13)))
    out = _gather_sc()(xyzp, gidx.reshape(-1), cpad)
    neighborhood = out.reshape(_B, _G, _K, 16)[..., :3]
    return (neighborhood, center)
